# Initial kernel scaffold; baseline (speedup 1.0000x reference)
import jax
import jax.numpy as jnp
from jax.experimental import pallas as pl


def kernel(x, table):
    raise NotImplementedError("write your pallas kernel here")


# speedup vs baseline: 1.0132x; 1.0132x over previous
import functools
import math

import jax
import jax.numpy as jnp
from jax import lax
from jax.experimental import pallas as pl
from jax.experimental.pallas import tpu as pltpu
from jax.experimental.pallas import tpu_sc as plsc

VOCAB_SIZE = 100000
D_MODEL = 1024
SCALE = math.sqrt(D_MODEL)

NC = 2
NS = 16
NW = NC * NS
LANES = 16

B_TOTAL = 4 * 4096
B_PER_W = B_TOTAL // NW
CHUNK = 64
N_CHUNKS = B_PER_W // CHUNK
SLICES_PER_ROW = D_MODEL // LANES


def _emb_kernel(x_hbm, table_hbm, out_hbm, idx_v, rows_v, sem):
    wid = lax.axis_index("s") * NC + lax.axis_index("c")
    base = wid * B_PER_W

    pltpu.sync_copy(x_hbm.at[wid], idx_v)

    for g in range(N_CHUNKS):
        pltpu.async_copy(table_hbm.at[idx_v.at[g]], rows_v, sem).wait()

        def scale_row(r, _):
            for c in range(SLICES_PER_ROW):
                sl = pl.ds(c * LANES, LANES)
                rows_v[r, sl] = rows_v[r, sl] * SCALE
            return _

        lax.fori_loop(0, CHUNK, scale_row, None)

        pltpu.sync_copy(rows_v, out_hbm.at[pl.ds(base + g * CHUNK, CHUNK)])


@jax.jit
def _emb(x_flat, table):
    mesh = plsc.VectorSubcoreMesh(core_axis_name="c", subcore_axis_name="s")
    f = functools.partial(
        pl.kernel,
        mesh=mesh,
        out_type=jax.ShapeDtypeStruct((B_TOTAL, D_MODEL), jnp.float32),
        scratch_types=[
            pltpu.VMEM((N_CHUNKS, CHUNK), jnp.int32),
            pltpu.VMEM((CHUNK, D_MODEL), jnp.float32),
            pltpu.SemaphoreType.DMA,
        ],
    )(_emb_kernel)
    return f(x_flat.reshape(NW, N_CHUNKS, CHUNK), table)


def kernel(x, table):
    out = _emb(x.reshape(-1).astype(jnp.int32), table)
    return out.reshape(x.shape[0], x.shape[1], D_MODEL)


# speedup vs baseline: 1.3165x; 1.2993x over previous
import functools
import math

import jax
import jax.numpy as jnp
from jax import lax
from jax.experimental import pallas as pl
from jax.experimental.pallas import tpu as pltpu
from jax.experimental.pallas import tpu_sc as plsc

VOCAB_SIZE = 100000
D_MODEL = 1024
SCALE = math.sqrt(D_MODEL)

NC = 2
NS = 16
NW = NC * NS
LANES = 16

B_TOTAL = 4 * 4096
B_PER_W = B_TOTAL // NW
CHUNK = 32
N_CHUNKS = B_PER_W // CHUNK
SLICES_PER_ROW = D_MODEL // LANES


def _scale_chunk(buf):
    def scale_row(r, carry):
        for c in range(SLICES_PER_ROW):
            sl = pl.ds(c * LANES, LANES)
            buf[r, sl] = buf[r, sl] * SCALE
        return carry

    lax.fori_loop(0, CHUNK, scale_row, None)


def _emb_kernel(x_hbm, table_hbm, out_hbm, idx_v, rows_v, in_sems, out_sems):
    wid = lax.axis_index("s") * NC + lax.axis_index("c")
    base = wid * B_PER_W

    pltpu.sync_copy(x_hbm.at[wid], idx_v)

    def start_gather(g):
        b = g % 2
        return pltpu.async_copy(table_hbm.at[idx_v.at[g]], rows_v.at[b],
                                in_sems.at[b])

    def start_out(g):
        b = g % 2
        return pltpu.async_copy(rows_v.at[b],
                                out_hbm.at[pl.ds(base + g * CHUNK, CHUNK)],
                                out_sems.at[b])

    gathers = [start_gather(0)]
    outs = [None, None]
    for g in range(N_CHUNKS):
        if g + 1 < N_CHUNKS:
            if outs[(g + 1) % 2] is not None:
                outs[(g + 1) % 2].wait()
            gathers.append(start_gather(g + 1))
        gathers[g].wait()
        _scale_chunk(rows_v.at[g % 2])
        outs[g % 2] = start_out(g)

    outs[(N_CHUNKS - 2) % 2].wait()
    outs[(N_CHUNKS - 1) % 2].wait()


@jax.jit
def _emb(x_flat, table):
    mesh = plsc.VectorSubcoreMesh(core_axis_name="c", subcore_axis_name="s")
    f = functools.partial(
        pl.kernel,
        mesh=mesh,
        out_type=jax.ShapeDtypeStruct((B_TOTAL, D_MODEL), jnp.float32),
        scratch_types=[
            pltpu.VMEM((N_CHUNKS, CHUNK), jnp.int32),
            pltpu.VMEM((2, CHUNK, D_MODEL), jnp.float32),
            pltpu.SemaphoreType.DMA((2,)),
            pltpu.SemaphoreType.DMA((2,)),
        ],
    )(_emb_kernel)
    return f(x_flat.reshape(NW, N_CHUNKS, CHUNK), table)


def kernel(x, table):
    out = _emb(x.reshape(-1).astype(jnp.int32), table)
    return out.reshape(x.shape[0], x.shape[1], D_MODEL)


# speedup vs baseline: 1.4427x; 1.0959x over previous
import functools
import math

import jax
import jax.numpy as jnp
from jax import lax
from jax.experimental import pallas as pl
from jax.experimental.pallas import tpu as pltpu
from jax.experimental.pallas import tpu_sc as plsc

VOCAB_SIZE = 100000
D_MODEL = 1024
SCALE = math.sqrt(D_MODEL)

NC = 2
NS = 16
NW = NC * NS
LANES = 16

B_TOTAL = 4 * 4096
B_PER_W = B_TOTAL // NW
CHUNK = 32
N_CHUNKS = B_PER_W // CHUNK
NBUF = 3
SLICES_PER_ROW = D_MODEL // LANES


def _scale_chunk(buf):
    @plsc.parallel_loop(0, CHUNK)
    def scale_row(r):
        for c in range(SLICES_PER_ROW):
            sl = pl.ds(c * LANES, LANES)
            buf[r, sl] = buf[r, sl] * SCALE


def _emb_kernel(x_hbm, table_hbm, out_hbm, idx_v, rows_v, in_sems, out_sems):
    wid = lax.axis_index("s") * NC + lax.axis_index("c")
    base = wid * B_PER_W

    pltpu.sync_copy(x_hbm.at[wid], idx_v)

    def start_gather(g):
        b = g % NBUF
        return pltpu.async_copy(table_hbm.at[idx_v.at[g]], rows_v.at[b],
                                in_sems.at[b])

    def start_out(g):
        b = g % NBUF
        return pltpu.async_copy(rows_v.at[b],
                                out_hbm.at[pl.ds(base + g * CHUNK, CHUNK)],
                                out_sems.at[b])

    gathers = [start_gather(g) for g in range(NBUF - 1)]
    outs = [None] * NBUF
    for g in range(N_CHUNKS):
        p = g + NBUF - 1
        if p < N_CHUNKS:
            b = p % NBUF
            if outs[b] is not None:
                outs[b].wait()
                outs[b] = None
            gathers.append(start_gather(p))
        gathers[g].wait()
        _scale_chunk(rows_v.at[g % NBUF])
        outs[g % NBUF] = start_out(g)

    for o in outs:
        if o is not None:
            o.wait()


@jax.jit
def _emb(x_flat, table):
    mesh = plsc.VectorSubcoreMesh(core_axis_name="c", subcore_axis_name="s")
    f = functools.partial(
        pl.kernel,
        mesh=mesh,
        out_type=jax.ShapeDtypeStruct((B_TOTAL, D_MODEL), jnp.float32),
        scratch_types=[
            pltpu.VMEM((N_CHUNKS, CHUNK), jnp.int32),
            pltpu.VMEM((NBUF, CHUNK, D_MODEL), jnp.float32),
            pltpu.SemaphoreType.DMA((NBUF,)),
            pltpu.SemaphoreType.DMA((NBUF,)),
        ],
    )(_emb_kernel)
    return f(x_flat.reshape(NW, N_CHUNKS, CHUNK), table)


def kernel(x, table):
    out = _emb(x.reshape(-1).astype(jnp.int32), table)
    return out.reshape(x.shape[0], x.shape[1], D_MODEL)


# speedup vs baseline: 1.5738x; 1.0909x over previous
import functools
import math

import jax
import jax.numpy as jnp
from jax import lax
from jax.experimental import pallas as pl
from jax.experimental.pallas import tpu as pltpu
from jax.experimental.pallas import tpu_sc as plsc

VOCAB_SIZE = 100000
D_MODEL = 1024
SCALE = math.sqrt(D_MODEL)

NC = 2
NS = 16
NW = NC * NS
LANES = 16

B_TOTAL = 4 * 4096
B_PER_W = B_TOTAL // NW
CHUNK = 16
N_CHUNKS = B_PER_W // CHUNK
NBUF = 6
SLICES_PER_ROW = D_MODEL // LANES


def _scale_chunk(buf):
    @plsc.parallel_loop(0, CHUNK)
    def scale_row(r):
        @plsc.parallel_loop(0, SLICES_PER_ROW, unroll=16)
        def scale_slice(c):
            sl = pl.ds(c * LANES, LANES)
            buf[r, sl] = buf[r, sl] * SCALE


def _emb_kernel(x_hbm, table_hbm, out_hbm, idx_v, rows_v, in_sems, out_sems):
    wid = lax.axis_index("s") * NC + lax.axis_index("c")
    base = wid * B_PER_W

    pltpu.sync_copy(x_hbm.at[wid], idx_v)

    def start_gather(g):
        b = g % NBUF
        return pltpu.async_copy(table_hbm.at[idx_v.at[g]], rows_v.at[b],
                                in_sems.at[b])

    def start_out(g):
        b = g % NBUF
        return pltpu.async_copy(rows_v.at[b],
                                out_hbm.at[pl.ds(base + g * CHUNK, CHUNK)],
                                out_sems.at[b])

    gathers = [start_gather(g) for g in range(NBUF - 1)]
    outs = [None] * NBUF
    for g in range(N_CHUNKS):
        p = g + NBUF - 1
        if p < N_CHUNKS:
            b = p % NBUF
            if outs[b] is not None:
                outs[b].wait()
                outs[b] = None
            gathers.append(start_gather(p))
        gathers[g].wait()
        _scale_chunk(rows_v.at[g % NBUF])
        outs[g % NBUF] = start_out(g)

    for o in outs:
        if o is not None:
            o.wait()


@jax.jit
def _emb(x_flat, table):
    mesh = plsc.VectorSubcoreMesh(core_axis_name="c", subcore_axis_name="s")
    f = functools.partial(
        pl.kernel,
        mesh=mesh,
        out_type=jax.ShapeDtypeStruct((B_TOTAL, D_MODEL), jnp.float32),
        scratch_types=[
            pltpu.VMEM((N_CHUNKS, CHUNK), jnp.int32),
            pltpu.VMEM((NBUF, CHUNK, D_MODEL), jnp.float32),
            pltpu.SemaphoreType.DMA((NBUF,)),
            pltpu.SemaphoreType.DMA((NBUF,)),
        ],
    )(_emb_kernel)
    return f(x_flat.reshape(NW, N_CHUNKS, CHUNK), table)


def kernel(x, table):
    out = _emb(x.reshape(-1).astype(jnp.int32), table)
    return out.reshape(x.shape[0], x.shape[1], D_MODEL)
